# baseline (speedup 1.0000x reference)
import functools

import jax
import jax.numpy as jnp
from jax import lax
from jax.experimental import pallas as pl
from jax.experimental.pallas import tpu as pltpu
from jax.experimental.pallas import tpu_sc as plsc

N_CELLS = 512
N_GENES = 1000
D = 128

_info = plsc.get_sparse_core_info()
_NC, _NS = _info.num_cores, _info.num_subcores
_BPW = 40
_NACT = N_GENES // _BPW


def _gather_sc(weight1, bias1, idx):
    mesh = plsc.VectorSubcoreMesh(core_axis_name="c", subcore_axis_name="s")

    @functools.partial(
        pl.kernel,
        mesh=mesh,
        out_type=(jax.ShapeDtypeStruct((N_GENES, D), jnp.float32),
                  jax.ShapeDtypeStruct((N_GENES,), jnp.float32)),
        scratch_types=[
            pltpu.VMEM((_BPW,), jnp.int32),
            pltpu.VMEM((_BPW, D), jnp.float32),
            pltpu.VMEM((_BPW,), jnp.float32),
            pltpu.SemaphoreType.DMA,
            pltpu.SemaphoreType.DMA,
            pltpu.SemaphoreType.DMA,
        ],
    )
    def k(w_hbm, b_hbm, idx_hbm, wout_hbm, bout_hbm,
          idx_v, rows_v, bsel_v, semw, semb, semo):
        wid = lax.axis_index("s") * _NC + lax.axis_index("c")
        base = wid * _BPW

        @pl.when(wid < _NACT)
        def _():
            pltpu.sync_copy(idx_hbm.at[pl.ds(base, _BPW)], idx_v)
            cw = pltpu.async_copy(w_hbm.at[idx_v], rows_v, semw)
            cb = pltpu.async_copy(b_hbm.at[idx_v], bsel_v, semb)
            cw.wait()
            ow = pltpu.async_copy(rows_v, wout_hbm.at[pl.ds(base, _BPW)], semo)
            cb.wait()
            pltpu.sync_copy(bsel_v, bout_hbm.at[pl.ds(base, _BPW)])
            ow.wait()

    return k(weight1, bias1, idx)


_CB = 32


def _tc_body(w_ref, b_ref, e_ref, out_ref):
    prod = e_ref[...] * w_ref[...][None, :, :]
    out_ref[...] = jnp.sum(jnp.swapaxes(prod, 1, 2), axis=1) + b_ref[...]


def kernel(cell_gene_embedding, gene_ix, weight1, bias1):
    w_gath, b_gath = _gather_sc(weight1, bias1, gene_ix)
    b2 = b_gath.reshape(1, N_GENES)

    out = pl.pallas_call(
        _tc_body,
        grid=(N_CELLS // _CB,),
        in_specs=[
            pl.BlockSpec((N_GENES, D), lambda i: (0, 0)),
            pl.BlockSpec((1, N_GENES), lambda i: (0, 0)),
            pl.BlockSpec((_CB, N_GENES, D), lambda i: (i, 0, 0)),
        ],
        out_specs=pl.BlockSpec((_CB, N_GENES), lambda i: (i, 0)),
        out_shape=jax.ShapeDtypeStruct((N_CELLS, N_GENES), jnp.float32),
    )(w_gath, b2, cell_gene_embedding)
    return out
